# baseline (speedup 1.0000x reference)
import jax
import jax.numpy as jnp
from jax.experimental import pallas as pl
from jax.experimental.pallas import tpu as pltpu

_BM = 400


def _ngcf_fused(adj_ref, xbf_ref, xblk_ref, w1t_ref, w2t_ref, b1_ref, b2_ref,
                out_ref):
    a = adj_ref[...].astype(jnp.bfloat16)
    neigh = jnp.dot(a, xbf_ref[...], preferred_element_type=jnp.float32)
    inter = xblk_ref[...] * neigh
    h = (jnp.dot(neigh, w1t_ref[...],
                 precision=jax.lax.Precision.HIGHEST,
                 preferred_element_type=jnp.float32)
         + jnp.dot(inter, w2t_ref[...],
                   precision=jax.lax.Precision.HIGHEST,
                   preferred_element_type=jnp.float32)
         + b1_ref[...] + b2_ref[...])
    out_ref[...] = jnp.where(h >= 0, h, 0.2 * h)


def kernel(x, adj_matrix, W1, b1, W2, b2):
    n, d = x.shape
    d_out = W1.shape[0]
    x_bf = x.astype(jnp.bfloat16)
    grid = (n // _BM,)
    return pl.pallas_call(
        _ngcf_fused,
        grid=grid,
        in_specs=[
            pl.BlockSpec((_BM, n), lambda i: (i, 0)),
            pl.BlockSpec((n, d), lambda i: (0, 0)),
            pl.BlockSpec((_BM, d), lambda i: (i, 0)),
            pl.BlockSpec((d, d_out), lambda i: (0, 0)),
            pl.BlockSpec((d, d_out), lambda i: (0, 0)),
            pl.BlockSpec((1, d_out), lambda i: (0, 0)),
            pl.BlockSpec((1, d_out), lambda i: (0, 0)),
        ],
        out_specs=pl.BlockSpec((_BM, d_out), lambda i: (i, 0)),
        out_shape=jax.ShapeDtypeStruct((n, d_out), jnp.float32),
        compiler_params=pltpu.CompilerParams(
            dimension_semantics=("parallel",),
            vmem_limit_bytes=100 * 1024 * 1024,
        ),
    )(adj_matrix, x_bf, x, W1.T, W2.T, b1.reshape(1, -1), b2.reshape(1, -1))


# speedup vs baseline: 1.0254x; 1.0254x over previous
import jax
import jax.numpy as jnp
from jax.experimental import pallas as pl
from jax.experimental.pallas import tpu as pltpu

_BM = 200


def _ngcf_fused(adj_ref, xbf_ref, w1t_ref, w2t_ref, b1_ref, b2_ref,
                out_ref):
    i = pl.program_id(0)
    a = adj_ref[...].astype(jnp.bfloat16)
    neigh = jnp.dot(a, xbf_ref[...], preferred_element_type=jnp.float32)
    xblk = xbf_ref[pl.ds(i * _BM, _BM), :].astype(jnp.float32)
    inter = xblk * neigh
    h = (jnp.dot(neigh.astype(jnp.bfloat16), w1t_ref[...],
                 preferred_element_type=jnp.float32)
         + jnp.dot(inter.astype(jnp.bfloat16), w2t_ref[...],
                   preferred_element_type=jnp.float32)
         + b1_ref[...] + b2_ref[...])
    out_ref[...] = jnp.where(h >= 0, h, 0.2 * h)


def kernel(x, adj_matrix, W1, b1, W2, b2):
    n, d = x.shape
    d_out = W1.shape[0]
    x_bf = x.astype(jnp.bfloat16)
    w_bf = jnp.concatenate([W1.T, W2.T], axis=0).astype(jnp.bfloat16)
    grid = (n // _BM,)
    return pl.pallas_call(
        _ngcf_fused,
        grid=grid,
        in_specs=[
            pl.BlockSpec((_BM, n), lambda i: (i, 0)),
            pl.BlockSpec((n, d), lambda i: (0, 0)),
            pl.BlockSpec((d, d_out), lambda i: (0, 0)),
            pl.BlockSpec((d, d_out), lambda i: (1, 0)),
            pl.BlockSpec((1, d_out), lambda i: (0, 0)),
            pl.BlockSpec((1, d_out), lambda i: (0, 0)),
        ],
        out_specs=pl.BlockSpec((_BM, d_out), lambda i: (i, 0)),
        out_shape=jax.ShapeDtypeStruct((n, d_out), jnp.float32),
        compiler_params=pltpu.CompilerParams(
            dimension_semantics=("parallel",),
            vmem_limit_bytes=100 * 1024 * 1024,
        ),
    )(adj_matrix, x_bf, w_bf, w_bf, b1.reshape(1, -1), b2.reshape(1, -1))


# speedup vs baseline: 1.0550x; 1.0288x over previous
import jax
import jax.numpy as jnp
from jax.experimental import pallas as pl
from jax.experimental.pallas import tpu as pltpu

_BM = 400


def _ngcf_fused(adj_ref, xbf_ref, w1t_ref, w2t_ref, b1_ref, b2_ref,
                out_ref):
    i = pl.program_id(0)
    a = adj_ref[...].astype(jnp.bfloat16)
    neigh = jnp.dot(a, xbf_ref[...], preferred_element_type=jnp.float32)
    xblk = xbf_ref[pl.ds(i * _BM, _BM), :].astype(jnp.float32)
    inter = xblk * neigh
    h = (jnp.dot(neigh.astype(jnp.bfloat16), w1t_ref[...],
                 preferred_element_type=jnp.float32)
         + jnp.dot(inter.astype(jnp.bfloat16), w2t_ref[...],
                   preferred_element_type=jnp.float32)
         + b1_ref[...] + b2_ref[...])
    out_ref[...] = jnp.where(h >= 0, h, 0.2 * h)


def kernel(x, adj_matrix, W1, b1, W2, b2):
    n, d = x.shape
    d_out = W1.shape[0]
    x_bf = x.astype(jnp.bfloat16)
    w_bf = jnp.concatenate([W1.T, W2.T], axis=0).astype(jnp.bfloat16)
    grid = (n // _BM,)
    return pl.pallas_call(
        _ngcf_fused,
        grid=grid,
        in_specs=[
            pl.BlockSpec((_BM, n), lambda i: (i, 0)),
            pl.BlockSpec((n, d), lambda i: (0, 0)),
            pl.BlockSpec((d, d_out), lambda i: (0, 0)),
            pl.BlockSpec((d, d_out), lambda i: (1, 0)),
            pl.BlockSpec((1, d_out), lambda i: (0, 0)),
            pl.BlockSpec((1, d_out), lambda i: (0, 0)),
        ],
        out_specs=pl.BlockSpec((_BM, d_out), lambda i: (i, 0)),
        out_shape=jax.ShapeDtypeStruct((n, d_out), jnp.float32),
        compiler_params=pltpu.CompilerParams(
            dimension_semantics=("parallel",),
            vmem_limit_bytes=100 * 1024 * 1024,
        ),
    )(adj_matrix, x_bf, w_bf, w_bf, b1.reshape(1, -1), b2.reshape(1, -1))


# speedup vs baseline: 1.0971x; 1.0399x over previous
import jax
import jax.numpy as jnp
from jax.experimental import pallas as pl
from jax.experimental.pallas import tpu as pltpu

_BM = 400


def _ngcf_fused(adj_ref, x_ref, w1_ref, w2_ref, b1_ref, b2_ref,
                out_ref, xbf_s, w1t_s, w2t_s):
    i = pl.program_id(0)

    @pl.when(i == 0)
    def _init():
        xbf_s[...] = x_ref[...].astype(jnp.bfloat16)
        w1t_s[...] = w1_ref[...].T.astype(jnp.bfloat16)
        w2t_s[...] = w2_ref[...].T.astype(jnp.bfloat16)

    a = adj_ref[...].astype(jnp.bfloat16)
    neigh = jnp.dot(a, xbf_s[...], preferred_element_type=jnp.float32)
    xblk = x_ref[pl.ds(i * _BM, _BM), :]
    inter = xblk * neigh
    h = (jnp.dot(neigh.astype(jnp.bfloat16), w1t_s[...],
                 preferred_element_type=jnp.float32)
         + jnp.dot(inter.astype(jnp.bfloat16), w2t_s[...],
                   preferred_element_type=jnp.float32)
         + b1_ref[...] + b2_ref[...])
    out_ref[...] = jnp.where(h >= 0, h, 0.2 * h)


def kernel(x, adj_matrix, W1, b1, W2, b2):
    n, d = x.shape
    d_out = W1.shape[0]
    grid = (n // _BM,)
    return pl.pallas_call(
        _ngcf_fused,
        grid=grid,
        in_specs=[
            pl.BlockSpec((_BM, n), lambda i: (i, 0)),
            pl.BlockSpec((n, d), lambda i: (0, 0)),
            pl.BlockSpec((d_out, d), lambda i: (0, 0)),
            pl.BlockSpec((d_out, d), lambda i: (0, 0)),
            pl.BlockSpec((1, d_out), lambda i: (0, 0)),
            pl.BlockSpec((1, d_out), lambda i: (0, 0)),
        ],
        out_specs=pl.BlockSpec((_BM, d_out), lambda i: (i, 0)),
        out_shape=jax.ShapeDtypeStruct((n, d_out), jnp.float32),
        scratch_shapes=[
            pltpu.VMEM((n, d), jnp.bfloat16),
            pltpu.VMEM((d, d_out), jnp.bfloat16),
            pltpu.VMEM((d, d_out), jnp.bfloat16),
        ],
        compiler_params=pltpu.CompilerParams(
            dimension_semantics=("arbitrary",),
            vmem_limit_bytes=100 * 1024 * 1024,
        ),
    )(adj_matrix, x, W1, W2, b1.reshape(1, -1), b2.reshape(1, -1))
